# Initial kernel scaffold; baseline (speedup 1.0000x reference)
import jax
import jax.numpy as jnp
from jax.experimental import pallas as pl


def kernel(index_list, ent_emb, rel_emb, w_r_w, w_r_b, w_u2r_w, w_u2r_b, w_u_w, w_u_b, w_r2u_w, w_r2u_b):
    raise NotImplementedError("write your pallas kernel here")


# speedup vs baseline: 3.1401x; 3.1401x over previous
import functools

import jax
import jax.numpy as jnp
from jax import lax
from jax.experimental import pallas as pl
from jax.experimental.pallas import tpu as pltpu
from jax.experimental.pallas import tpu_sc as plsc

NE = 10000
NR = 10000
NP = 10240
H = 128
E = 320000
NC = 2
NS = 16
NW = NC * NS
K = 128
HR = NP // NC
TRASH = HR
AR = HR + 8
WPT = HR // NS
CH = 157
EPT = CH * K
EPP = EPT * NS
CPR = K // 16
CW = 16
ZR = 80
CV = 5152

_mesh = plsc.VectorSubcoreMesh(core_axis_name="c", subcore_axis_name="s")


def _zero_vmem(ref, rows, width):
    z = jnp.zeros((16,), jnp.float32)
    cpr = width // 16

    def st(i, carry):
        ref[i // cpr, pl.ds((i % cpr) * 16, 16)] = z
        return carry

    lax.fori_loop(0, rows * cpr, st, 0)


@functools.partial(
    pl.kernel,
    mesh=_mesh,
    out_type=jax.ShapeDtypeStruct((NP, H), jnp.float32),
    scratch_types=[
        pltpu.VMEM((CH, K), jnp.int32),
        pltpu.VMEM((CH, K), jnp.int32),
        pltpu.VMEM((K,), jnp.int32),
        pltpu.VMEM((K,), jnp.int32),
        pltpu.VMEM((ZR,), jnp.int32),
        pltpu.VMEM((K, H), jnp.float32),
        pltpu.VMEM((ZR, H), jnp.float32),
        pltpu.VMEM_SHARED((AR, H), jnp.float32),
        pltpu.SemaphoreType.DMA,
    ],
)
def _sc_seg_sum(src_hbm, dst_hbm, tab_hbm, out_hbm,
                sidx_v, didx_v, sidx_c, didx_c, ridx_c, rows_v,
                zbuf, acc_sh, sem):
    cid = lax.axis_index("c")
    sid = lax.axis_index("s")

    _zero_vmem(zbuf, ZR, H)

    pltpu.sync_copy(src_hbm.at[sid], sidx_v)
    pltpu.sync_copy(dst_hbm.at[sid], didx_v)

    lanes = jnp.arange(16, dtype=jnp.int32)

    def fill_ridx(start):
        def fi(i, carry):
            ridx_c[pl.ds(i * 16, 16)] = start + i * 16 + lanes
            return carry

        lax.fori_loop(0, ZR // 16, fi, 0)

    def zchunk(j, carry):
        fill_ridx(sid * WPT + j * ZR)
        pltpu.sync_copy(zbuf, acc_sh.at[ridx_c])
        return carry

    lax.fori_loop(0, WPT // ZR, zchunk, 0)
    plsc.subcore_barrier()

    base = cid * HR

    def chunk(c, carry):
        def prep(i, carry2):
            col = pl.ds(i * 16, 16)
            sidx_c[col] = sidx_v[c, col]
            d = didx_v[c, col] - base
            ok = jnp.logical_and(d >= 0, d < HR)
            didx_c[col] = jnp.where(ok, d, TRASH)
            return carry2

        lax.fori_loop(0, CPR, prep, 0)
        pltpu.async_copy(tab_hbm.at[sidx_c], rows_v, sem).wait()
        pltpu.sync_copy(rows_v, acc_sh.at[didx_c], add=True)
        return carry

    lax.fori_loop(0, CH, chunk, 0)
    plsc.subcore_barrier()

    def wchunk(j, carry):
        fill_ridx(sid * WPT + j * ZR)
        pltpu.async_copy(acc_sh.at[ridx_c], zbuf, sem).wait()
        pltpu.sync_copy(
            zbuf, out_hbm.at[pl.ds(cid * HR + sid * WPT + j * ZR, ZR)])
        return carry

    lax.fori_loop(0, WPT // ZR, wchunk, 0)


BR = 1024


def _tc_update_body(p_ref, c_ref, told_ref, wm_ref, bm_ref, ws_ref, bs_ref,
                    upd_ref, nupd_ref):
    cnt = c_ref[:, 0:1]
    mean = p_ref[...] / jnp.maximum(cnt, 1.0)
    msg = lax.dot_general(mean, wm_ref[...], (((1,), (1,)), ((), ())),
                          preferred_element_type=jnp.float32) + bm_ref[...]
    msg = jnp.maximum(msg, 0.0)
    told = told_ref[...]
    own = lax.dot_general(told, ws_ref[...], (((1,), (1,)), ((), ())),
                          preferred_element_type=jnp.float32) + bs_ref[...]
    upd = jnp.where(cnt > 0.0, 0.5 * msg + 0.5 * own, told)
    upd_ref[...] = upd
    norm = jnp.sqrt(jnp.sum(upd * upd, axis=1, keepdims=True))
    nupd_ref[...] = upd / jnp.maximum(norm, 1e-12)


def _tc_update(sums, cnts, told, wm, bm, ws, bs):
    n = told.shape[0]
    full = pl.BlockSpec((BR, H), lambda i: (i, 0))
    return pl.pallas_call(
        _tc_update_body,
        grid=(n // BR,),
        in_specs=[
            full,
            pl.BlockSpec((BR, CW), lambda i: (i, 0)),
            full,
            pl.BlockSpec((H, H), lambda i: (0, 0)),
            pl.BlockSpec((1, H), lambda i: (0, 0)),
            pl.BlockSpec((H, H), lambda i: (0, 0)),
            pl.BlockSpec((1, H), lambda i: (0, 0)),
        ],
        out_specs=(full, full),
        out_shape=(jax.ShapeDtypeStruct((n, H), jnp.float32),
                   jax.ShapeDtypeStruct((n, H), jnp.float32)),
    )(sums, cnts, told, wm, bm.reshape(1, H), ws, bs.reshape(1, H))


def _tc_norm_body(x_ref, o_ref):
    x = x_ref[...]
    norm = jnp.sqrt(jnp.sum(x * x, axis=1, keepdims=True))
    o_ref[...] = x / jnp.maximum(norm, 1e-12)


def _tc_norm(x):
    n = x.shape[0]
    return pl.pallas_call(
        _tc_norm_body,
        grid=(n // BR,),
        in_specs=[pl.BlockSpec((BR, H), lambda i: (i, 0))],
        out_specs=pl.BlockSpec((BR, H), lambda i: (i, 0)),
        out_shape=jax.ShapeDtypeStruct((n, H), jnp.float32),
    )(x)


def kernel(index_list, ent_emb, rel_emb, w_r_w, w_r_b, w_u2r_w, w_u2r_b,
           w_u_w, w_u_b, w_r2u_w, w_r2u_b):
    u_flat = index_list[0, 0]
    r_flat = index_list[0, 1]
    pad_src = jnp.zeros((EPP - E,), jnp.int32)
    pad_dst = jnp.full((EPP - E,), NP, jnp.int32)
    u_src = jnp.concatenate([u_flat, pad_src]).reshape(NS, CH, K)
    u_dst = jnp.concatenate([u_flat, pad_dst]).reshape(NS, CH, K)
    r_src = jnp.concatenate([r_flat, pad_src]).reshape(NS, CH, K)
    r_dst = jnp.concatenate([r_flat, pad_dst]).reshape(NS, CH, K)

    pad_e = jnp.zeros((NP - NE, H), jnp.float32)
    pad_r = jnp.zeros((NP - NR, H), jnp.float32)
    g_h = _tc_norm(jnp.concatenate([ent_emb, pad_e], axis=0))
    g_h_0 = _tc_norm(jnp.concatenate([rel_emb, pad_r], axis=0))

    ones_tab = jnp.ones((NP, H), jnp.float32)
    xs = (
        jnp.stack([u_src, r_src, u_src, r_src, u_src, r_src]),
        jnp.stack([r_dst, u_dst, r_dst, u_dst, r_dst, u_dst]),
        jnp.stack([w_u2r_w, w_r2u_w, w_u2r_w, w_r2u_w, w_u2r_w, w_r2u_w]),
        jnp.stack([w_u2r_b, w_r2u_b, w_u2r_b, w_r2u_b, w_u2r_b, w_r2u_b]),
        jnp.stack([w_r_w, w_u_w, w_r_w, w_u_w, w_r_w, w_u_w]),
        jnp.stack([w_r_b, w_u_b, w_r_b, w_u_b, w_r_b, w_u_b]),
        jnp.array([False, True, False, True, False, True]),
        jnp.array([True, True, False, False, False, False]),
    )

    def body(carry, x):
        g_h_c, g_h_0_c, last_upd, cnt_r_c, cnt_u_c = carry
        src3, dst3, wm, bm, ws, bs, ent_phase, cnt_phase = x
        table = jnp.where(cnt_phase, ones_tab,
                          jnp.where(ent_phase, last_upd, g_h_c))
        told = jnp.where(ent_phase, g_h_c, g_h_0_c)
        sums = _sc_seg_sum(src3, dst3, table)
        cntmat = sums[:, :CW]
        cnt_r_n = jnp.where(jnp.logical_and(cnt_phase, ~ent_phase),
                            cntmat, cnt_r_c)
        cnt_u_n = jnp.where(jnp.logical_and(cnt_phase, ent_phase),
                            cntmat, cnt_u_c)
        cnt = jnp.where(ent_phase, cnt_u_n, cnt_r_n)
        upd, nupd = _tc_update(sums, cnt, told, wm, bm, ws, bs)
        g_h_n = jnp.where(cnt_phase, g_h_c,
                          jnp.where(ent_phase, nupd, g_h_c))
        g_h_0_n = jnp.where(cnt_phase, g_h_0_c,
                            jnp.where(ent_phase, g_h_0_c, nupd))
        upd_n = jnp.where(cnt_phase, last_upd, upd)
        return (g_h_n, g_h_0_n, upd_n, cnt_r_n, cnt_u_n), None

    (g_h, g_h_0, _, _, _), _ = lax.scan(
        body,
        (g_h, g_h_0, jnp.zeros((NP, H), jnp.float32),
         jnp.zeros((NP, CW), jnp.float32), jnp.zeros((NP, CW), jnp.float32)),
        xs)

    return g_h[:NE]


# speedup vs baseline: 3.7234x; 1.1858x over previous
import functools

import jax
import jax.numpy as jnp
from jax import lax
from jax.experimental import pallas as pl
from jax.experimental.pallas import tpu as pltpu
from jax.experimental.pallas import tpu_sc as plsc

NE = 10000
NR = 10000
NP = 10240
H = 128
E = 320000
NC = 2
NS = 16
NW = NC * NS
K = 128
HR = NP // NC
TRASH = HR
AR = HR + 8
WPT = HR // NS
CH = 157
EPT = CH * K
EPP = EPT * NS
CPR = K // 16
CW = 16
ZR = 80
CV = 5152

_mesh = plsc.VectorSubcoreMesh(core_axis_name="c", subcore_axis_name="s")


def _zero_vmem(ref, rows, width):
    z = jnp.zeros((16,), jnp.float32)
    cpr = width // 16

    def st(i, carry):
        ref[i // cpr, pl.ds((i % cpr) * 16, 16)] = z
        return carry

    lax.fori_loop(0, rows * cpr, st, 0)


@functools.partial(
    pl.kernel,
    mesh=_mesh,
    out_type=jax.ShapeDtypeStruct((NP, H), jnp.float32),
    scratch_types=[
        pltpu.VMEM((CH, K), jnp.int32),
        pltpu.VMEM((CH, K), jnp.int32),
        pltpu.VMEM((K,), jnp.int32),
        pltpu.VMEM((K,), jnp.int32),
        pltpu.VMEM((K,), jnp.int32),
        pltpu.VMEM((K,), jnp.int32),
        pltpu.VMEM((ZR,), jnp.int32),
        pltpu.VMEM((K, H), jnp.float32),
        pltpu.VMEM((K, H), jnp.float32),
        pltpu.VMEM((ZR, H), jnp.float32),
        pltpu.VMEM_SHARED((AR, H), jnp.float32),
        pltpu.SemaphoreType.DMA,
        pltpu.SemaphoreType.DMA,
    ],
)
def _sc_seg_sum(src_hbm, dst_hbm, tab_hbm, out_hbm,
                sidx_v, didx_v, sidx_c, didx_c, sidx_c1, didx_c1, ridx_c,
                rows_v, rows_v1, zbuf, acc_sh, sem, sem1):
    cid = lax.axis_index("c")
    sid = lax.axis_index("s")

    _zero_vmem(zbuf, ZR, H)

    pltpu.sync_copy(src_hbm.at[sid], sidx_v)
    pltpu.sync_copy(dst_hbm.at[sid], didx_v)

    lanes = jnp.arange(16, dtype=jnp.int32)

    def fill_ridx(start):
        def fi(i, carry):
            ridx_c[pl.ds(i * 16, 16)] = start + i * 16 + lanes
            return carry

        lax.fori_loop(0, ZR // 16, fi, 0)

    def zchunk(j, carry):
        fill_ridx(sid * WPT + j * ZR)
        pltpu.sync_copy(zbuf, acc_sh.at[ridx_c])
        return carry

    lax.fori_loop(0, WPT // ZR, zchunk, 0)
    plsc.subcore_barrier()

    base = cid * HR

    def prep(c, sbuf, dbuf):
        def pi(i, carry2):
            col = pl.ds(i * 16, 16)
            sbuf[col] = sidx_v[c, col]
            d = didx_v[c, col] - base
            ok = jnp.logical_and(d >= 0, d < HR)
            dbuf[col] = jnp.where(ok, d, TRASH)
            return carry2

        lax.fori_loop(0, CPR, pi, 0)

    prep(0, sidx_c, didx_c)
    pltpu.async_copy(tab_hbm.at[sidx_c], rows_v, sem)

    def chunk2(p, carry):
        c = 1 + 2 * p
        prep(c, sidx_c1, didx_c1)
        pltpu.async_copy(tab_hbm.at[sidx_c1], rows_v1, sem1)
        pltpu.make_async_copy(tab_hbm.at[sidx_c], rows_v, sem).wait()
        pltpu.sync_copy(rows_v, acc_sh.at[didx_c], add=True)
        prep(c + 1, sidx_c, didx_c)
        pltpu.async_copy(tab_hbm.at[sidx_c], rows_v, sem)
        pltpu.make_async_copy(tab_hbm.at[sidx_c1], rows_v1, sem1).wait()
        pltpu.sync_copy(rows_v1, acc_sh.at[didx_c1], add=True)
        return carry

    lax.fori_loop(0, (CH - 1) // 2, chunk2, 0)
    pltpu.make_async_copy(tab_hbm.at[sidx_c], rows_v, sem).wait()
    pltpu.sync_copy(rows_v, acc_sh.at[didx_c], add=True)
    plsc.subcore_barrier()

    def wchunk(j, carry):
        fill_ridx(sid * WPT + j * ZR)
        pltpu.async_copy(acc_sh.at[ridx_c], zbuf, sem).wait()
        pltpu.sync_copy(
            zbuf, out_hbm.at[pl.ds(cid * HR + sid * WPT + j * ZR, ZR)])
        return carry

    lax.fori_loop(0, WPT // ZR, wchunk, 0)


BR = 1024


def _tc_update_body(p_ref, c_ref, told_ref, wm_ref, bm_ref, ws_ref, bs_ref,
                    upd_ref, nupd_ref):
    cnt = c_ref[:, 0:1]
    mean = p_ref[...] / jnp.maximum(cnt, 1.0)
    msg = lax.dot_general(mean, wm_ref[...], (((1,), (1,)), ((), ())),
                          preferred_element_type=jnp.float32) + bm_ref[...]
    msg = jnp.maximum(msg, 0.0)
    told = told_ref[...]
    own = lax.dot_general(told, ws_ref[...], (((1,), (1,)), ((), ())),
                          preferred_element_type=jnp.float32) + bs_ref[...]
    upd = jnp.where(cnt > 0.0, 0.5 * msg + 0.5 * own, told)
    upd_ref[...] = upd
    norm = jnp.sqrt(jnp.sum(upd * upd, axis=1, keepdims=True))
    nupd_ref[...] = upd / jnp.maximum(norm, 1e-12)


def _tc_update(sums, cnts, told, wm, bm, ws, bs):
    n = told.shape[0]
    full = pl.BlockSpec((BR, H), lambda i: (i, 0))
    return pl.pallas_call(
        _tc_update_body,
        grid=(n // BR,),
        in_specs=[
            full,
            pl.BlockSpec((BR, CW), lambda i: (i, 0)),
            full,
            pl.BlockSpec((H, H), lambda i: (0, 0)),
            pl.BlockSpec((1, H), lambda i: (0, 0)),
            pl.BlockSpec((H, H), lambda i: (0, 0)),
            pl.BlockSpec((1, H), lambda i: (0, 0)),
        ],
        out_specs=(full, full),
        out_shape=(jax.ShapeDtypeStruct((n, H), jnp.float32),
                   jax.ShapeDtypeStruct((n, H), jnp.float32)),
    )(sums, cnts, told, wm, bm.reshape(1, H), ws, bs.reshape(1, H))


def _tc_norm_body(x_ref, o_ref):
    x = x_ref[...]
    norm = jnp.sqrt(jnp.sum(x * x, axis=1, keepdims=True))
    o_ref[...] = x / jnp.maximum(norm, 1e-12)


def _tc_norm(x):
    n = x.shape[0]
    return pl.pallas_call(
        _tc_norm_body,
        grid=(n // BR,),
        in_specs=[pl.BlockSpec((BR, H), lambda i: (i, 0))],
        out_specs=pl.BlockSpec((BR, H), lambda i: (i, 0)),
        out_shape=jax.ShapeDtypeStruct((n, H), jnp.float32),
    )(x)


def kernel(index_list, ent_emb, rel_emb, w_r_w, w_r_b, w_u2r_w, w_u2r_b,
           w_u_w, w_u_b, w_r2u_w, w_r2u_b):
    u_flat = index_list[0, 0]
    r_flat = index_list[0, 1]
    pad_src = jnp.zeros((EPP - E,), jnp.int32)
    pad_dst = jnp.full((EPP - E,), NP, jnp.int32)
    u_src = jnp.concatenate([u_flat, pad_src]).reshape(NS, CH, K)
    u_dst = jnp.concatenate([u_flat, pad_dst]).reshape(NS, CH, K)
    r_src = jnp.concatenate([r_flat, pad_src]).reshape(NS, CH, K)
    r_dst = jnp.concatenate([r_flat, pad_dst]).reshape(NS, CH, K)

    pad_e = jnp.zeros((NP - NE, H), jnp.float32)
    pad_r = jnp.zeros((NP - NR, H), jnp.float32)
    g_h = _tc_norm(jnp.concatenate([ent_emb, pad_e], axis=0))
    g_h_0 = _tc_norm(jnp.concatenate([rel_emb, pad_r], axis=0))

    ones_tab = jnp.ones((NP, H), jnp.float32)
    xs = (
        jnp.stack([u_src, r_src, u_src, r_src, u_src, r_src]),
        jnp.stack([r_dst, u_dst, r_dst, u_dst, r_dst, u_dst]),
        jnp.stack([w_u2r_w, w_r2u_w, w_u2r_w, w_r2u_w, w_u2r_w, w_r2u_w]),
        jnp.stack([w_u2r_b, w_r2u_b, w_u2r_b, w_r2u_b, w_u2r_b, w_r2u_b]),
        jnp.stack([w_r_w, w_u_w, w_r_w, w_u_w, w_r_w, w_u_w]),
        jnp.stack([w_r_b, w_u_b, w_r_b, w_u_b, w_r_b, w_u_b]),
        jnp.array([False, True, False, True, False, True]),
        jnp.array([True, True, False, False, False, False]),
    )

    def body(carry, x):
        g_h_c, g_h_0_c, last_upd, cnt_r_c, cnt_u_c = carry
        src3, dst3, wm, bm, ws, bs, ent_phase, cnt_phase = x
        table = jnp.where(cnt_phase, ones_tab,
                          jnp.where(ent_phase, last_upd, g_h_c))
        told = jnp.where(ent_phase, g_h_c, g_h_0_c)
        sums = _sc_seg_sum(src3, dst3, table)
        cntmat = sums[:, :CW]
        cnt_r_n = jnp.where(jnp.logical_and(cnt_phase, ~ent_phase),
                            cntmat, cnt_r_c)
        cnt_u_n = jnp.where(jnp.logical_and(cnt_phase, ent_phase),
                            cntmat, cnt_u_c)
        cnt = jnp.where(ent_phase, cnt_u_n, cnt_r_n)
        upd, nupd = _tc_update(sums, cnt, told, wm, bm, ws, bs)
        g_h_n = jnp.where(cnt_phase, g_h_c,
                          jnp.where(ent_phase, nupd, g_h_c))
        g_h_0_n = jnp.where(cnt_phase, g_h_0_c,
                            jnp.where(ent_phase, g_h_0_c, nupd))
        upd_n = jnp.where(cnt_phase, last_upd, upd)
        return (g_h_n, g_h_0_n, upd_n, cnt_r_n, cnt_u_n), None

    (g_h, g_h_0, _, _, _), _ = lax.scan(
        body,
        (g_h, g_h_0, jnp.zeros((NP, H), jnp.float32),
         jnp.zeros((NP, CW), jnp.float32), jnp.zeros((NP, CW), jnp.float32)),
        xs)

    return g_h[:NE]
